# baseline (speedup 1.0000x reference)
import functools

import jax
import jax.numpy as jnp
from jax import lax
from jax.experimental import pallas as pl
from jax.experimental.pallas import tpu as pltpu
from jax.experimental.pallas import tpu_sc as plsc

_NW = 32
_LANES = 16
_N_AST = 100000


def _build_src_map(key_i32, val_i32, n_ast, n_cfg):
    m = key_i32.shape[0]
    assert m % _LANES == 0
    per = -(-n_ast // (_NW * _LANES)) * _LANES
    s_pad = per * _NW
    mesh = plsc.VectorSubcoreMesh(core_axis_name="c", subcore_axis_name="s")

    @functools.partial(
        pl.kernel,
        out_type=jax.ShapeDtypeStruct((s_pad,), jnp.int32),
        mesh=mesh,
        scratch_types=[
            pltpu.VMEM((m,), jnp.int32),
            pltpu.VMEM((m,), jnp.int32),
            pltpu.VMEM((per,), jnp.int32),
        ],
        compiler_params=pltpu.CompilerParams(needs_layout_passes=False),
    )
    def build(key_hbm, val_hbm, src_hbm, keys_v, vals_v, s_v):
        wid = lax.axis_index("s") * 2 + lax.axis_index("c")
        lo = wid * per

        def init_body(i, _):
            s_v[pl.ds(i * _LANES, _LANES)] = jnp.full((_LANES,), n_cfg, jnp.int32)
            return _

        lax.fori_loop(0, per // _LANES, init_body, None)

        pltpu.sync_copy(key_hbm, keys_v)
        pltpu.sync_copy(val_hbm, vals_v)

        def scan_body(i, _):
            k16 = keys_v[pl.ds(i * _LANES, _LANES)]
            inr = (k16 >= lo) & (k16 < lo + per)
            loc = jnp.where(inr, k16 - lo, 0)
            v16 = vals_v[pl.ds(i * _LANES, _LANES)]
            plsc.store_scatter(s_v, [loc], v16, mask=inr)
            return _

        lax.fori_loop(0, m // _LANES, scan_body, None)

        pltpu.sync_copy(s_v, src_hbm.at[pl.ds(lo, per)])

    return build(key_i32, val_i32)


def _lookup_rows(src_map, path_pad):
    nt_pad = path_pad.shape[0]
    s_pad = src_map.shape[0]
    per = nt_pad // _NW
    nrow = per // 128
    assert per * _NW == nt_pad and nrow * 128 == per
    mesh = plsc.VectorSubcoreMesh(core_axis_name="c", subcore_axis_name="s")

    @functools.partial(
        pl.kernel,
        out_type=jax.ShapeDtypeStruct((nt_pad // 128, 128), jnp.int32),
        mesh=mesh,
        scratch_types=[
            pltpu.VMEM((s_pad,), jnp.int32),
            pltpu.VMEM((per,), jnp.int32),
            pltpu.VMEM((nrow, 128), jnp.int32),
        ],
        compiler_params=pltpu.CompilerParams(needs_layout_passes=False),
    )
    def lookup(src_hbm, path_hbm, g_hbm, src_v, pidx_v, g_v):
        wid = lax.axis_index("s") * 2 + lax.axis_index("c")
        pltpu.sync_copy(src_hbm, src_v)
        pltpu.sync_copy(path_hbm.at[pl.ds(wid * per, per)], pidx_v)

        def body(r, _):
            for k in range(128 // _LANES):
                p16 = pidx_v[pl.ds(r * 128 + k * _LANES, _LANES)]
                g_v[r, pl.ds(k * _LANES, _LANES)] = plsc.load_gather(src_v, [p16])
            return _

        lax.fori_loop(0, nrow, body, None)
        pltpu.sync_copy(g_v, g_hbm.at[pl.ds(wid * nrow, nrow)])

    return lookup(src_map, path_pad)


def _gather_rows(g2d, cfg_ext):
    nr = g2d.shape[0]
    d = cfg_ext.shape[1]
    nrow = nr // _NW
    assert nrow * _NW == nr
    nbuf = 4
    assert nrow % nbuf == 0
    mesh = plsc.VectorSubcoreMesh(core_axis_name="c", subcore_axis_name="s")

    @functools.partial(
        pl.kernel,
        out_type=jax.ShapeDtypeStruct((nr * 128, d), jnp.float32),
        mesh=mesh,
        scratch_types=[
            pltpu.VMEM((nrow, 128), jnp.int32),
            pltpu.VMEM((nbuf, 128, d), jnp.float32),
            [pltpu.SemaphoreType.DMA] * nbuf,
        ],
        compiler_params=pltpu.CompilerParams(needs_layout_passes=False),
    )
    def gather(g_hbm, cfg_hbm, upd_hbm, g_v, rows_v, sems):
        wid = lax.axis_index("s") * 2 + lax.axis_index("c")
        base = wid * nrow * 128
        pltpu.sync_copy(g_hbm.at[pl.ds(wid * nrow, nrow)], g_v)

        def start(c, b):
            pltpu.async_copy(cfg_hbm.at[g_v.at[c]], rows_v.at[b], sems[b])

        def finish(c, b):
            pltpu.make_async_copy(
                cfg_hbm.at[g_v.at[c]], rows_v.at[b], sems[b]).wait()
            off = pl.multiple_of(c * 128, 128)
            pltpu.sync_copy(rows_v.at[b], upd_hbm.at[pl.ds(base + off, 128)])

        for b in range(nbuf):
            start(b, b)

        def ring_body(it, _):
            for b in range(nbuf):
                c = it * nbuf + b
                finish(c, b)

                @pl.when(c + nbuf < nrow)
                def _():
                    start(c + nbuf, b)

            return _

        lax.fori_loop(0, nrow // nbuf, ring_body, None)

    return gather(g2d, cfg_ext)


def _cat_project(prev2d, upd2d, w1, w2, b2d):
    nt, d = prev2d.shape
    blk = 2000
    assert nt % blk == 0

    def body(prev_ref, upd_ref, w1_ref, w2_ref, b_ref, out_ref):
        acc = jnp.dot(prev_ref[...], w1_ref[...], preferred_element_type=jnp.float32)
        acc += jnp.dot(upd_ref[...], w2_ref[...], preferred_element_type=jnp.float32)
        out_ref[...] = jnp.maximum(acc + b_ref[...], 0.0)

    return pl.pallas_call(
        body,
        grid=(nt // blk,),
        in_specs=[
            pl.BlockSpec((blk, d), lambda i: (i, 0)),
            pl.BlockSpec((blk, d), lambda i: (i, 0)),
            pl.BlockSpec((d, d), lambda i: (0, 0)),
            pl.BlockSpec((d, d), lambda i: (0, 0)),
            pl.BlockSpec((1, d), lambda i: (0, 0)),
        ],
        out_specs=pl.BlockSpec((blk, d), lambda i: (i, 0)),
        out_shape=jax.ShapeDtypeStruct((nt, d), jnp.float32),
    )(prev2d, upd2d, w1, w2, b2d)


def kernel(nr_ast_nodes, prev_nodes_occurrences, new_cfg_nodes_encodings,
           mapping_value_indices, mapping_key_indices, path_node_indices, W, b):
    p, l, d = prev_nodes_occurrences.shape
    n_cfg = new_cfg_nodes_encodings.shape[0]
    nt = p * l

    key_i32 = jnp.minimum(mapping_key_indices, nr_ast_nodes - 1).astype(jnp.int32)
    val_i32 = mapping_value_indices.astype(jnp.int32)
    nt_pad = -(-nt // (_NW * 128)) * _NW * 128
    path_pad = jnp.pad(path_node_indices.reshape(nt).astype(jnp.int32),
                       (0, nt_pad - nt))
    cfg_ext = jnp.concatenate(
        [new_cfg_nodes_encodings,
         jnp.zeros((8, d), new_cfg_nodes_encodings.dtype)], axis=0)

    src_map = _build_src_map(key_i32, val_i32, _N_AST, n_cfg)
    g2d = _lookup_rows(src_map, path_pad)
    upd2d = _gather_rows(g2d, cfg_ext)

    prev2d = prev_nodes_occurrences.reshape(nt, d)
    out2d = _cat_project(prev2d, upd2d, W[:d], W[d:], b.reshape(1, d))
    return out2d.reshape(p, l, d)


# speedup vs baseline: 11.9803x; 11.9803x over previous
import functools

import jax
import jax.numpy as jnp
from jax import lax
from jax.experimental import pallas as pl
from jax.experimental.pallas import tpu as pltpu
from jax.experimental.pallas import tpu_sc as plsc

_NW = 32
_LANES = 16
_N_AST = 100000


def _build_src_map(key_i32, val_i32, n_ast, n_cfg):
    m = key_i32.shape[0]
    assert m % _LANES == 0
    per = -(-n_ast // (_NW * _LANES)) * _LANES
    s_pad = per * _NW
    mesh = plsc.VectorSubcoreMesh(core_axis_name="c", subcore_axis_name="s")

    @functools.partial(
        pl.kernel,
        out_type=jax.ShapeDtypeStruct((s_pad,), jnp.int32),
        mesh=mesh,
        scratch_types=[
            pltpu.VMEM((m,), jnp.int32),
            pltpu.VMEM((m,), jnp.int32),
            pltpu.VMEM((per,), jnp.int32),
        ],
        compiler_params=pltpu.CompilerParams(needs_layout_passes=False),
    )
    def build(key_hbm, val_hbm, src_hbm, keys_v, vals_v, s_v):
        wid = lax.axis_index("s") * 2 + lax.axis_index("c")
        lo = wid * per

        def init_body(i, _):
            s_v[pl.ds(i * _LANES, _LANES)] = jnp.full((_LANES,), n_cfg, jnp.int32)
            return _

        lax.fori_loop(0, per // _LANES, init_body, None)

        pltpu.sync_copy(key_hbm, keys_v)
        pltpu.sync_copy(val_hbm, vals_v)

        def scan_body(i, _):
            k16 = keys_v[pl.ds(i * _LANES, _LANES)]
            inr = (k16 >= lo) & (k16 < lo + per)
            loc = jnp.where(inr, k16 - lo, 0)
            v16 = vals_v[pl.ds(i * _LANES, _LANES)]
            plsc.store_scatter(s_v, [loc], v16, mask=inr)
            return _

        lax.fori_loop(0, m // _LANES, scan_body, None)

        pltpu.sync_copy(s_v, src_hbm.at[pl.ds(lo, per)])

    return build(key_i32, val_i32)


def _lookup_rows(src_map, path_pad):
    nt_pad = path_pad.shape[0]
    s_pad = src_map.shape[0]
    per = nt_pad // _NW
    nrow = per // 128
    assert per * _NW == nt_pad and nrow * 128 == per
    mesh = plsc.VectorSubcoreMesh(core_axis_name="c", subcore_axis_name="s")

    @functools.partial(
        pl.kernel,
        out_type=jax.ShapeDtypeStruct((nt_pad // 128, 128), jnp.int32),
        mesh=mesh,
        scratch_types=[
            pltpu.VMEM((s_pad,), jnp.int32),
            pltpu.VMEM((per,), jnp.int32),
            pltpu.VMEM((nrow, 128), jnp.int32),
        ],
        compiler_params=pltpu.CompilerParams(needs_layout_passes=False),
    )
    def lookup(src_hbm, path_hbm, g_hbm, src_v, pidx_v, g_v):
        wid = lax.axis_index("s") * 2 + lax.axis_index("c")
        pltpu.sync_copy(src_hbm, src_v)
        pltpu.sync_copy(path_hbm.at[pl.ds(wid * per, per)], pidx_v)

        def body(r, _):
            for k in range(128 // _LANES):
                p16 = pidx_v[pl.ds(r * 128 + k * _LANES, _LANES)]
                g_v[r, pl.ds(k * _LANES, _LANES)] = plsc.load_gather(src_v, [p16])
            return _

        lax.fori_loop(0, nrow, body, None)
        pltpu.sync_copy(g_v, g_hbm.at[pl.ds(wid * nrow, nrow)])

    return lookup(src_map, path_pad)


def _gather_cat_project(prev2d, g_row, cfg_ext, w1, w2, b2d):
    nt, d = prev2d.shape
    nc = cfg_ext.shape[0]
    blk = 2000
    assert nt % blk == 0

    def body(g_ref, prev_ref, cfg_ref, w1_ref, w2_ref, b_ref, out_ref, upd_ref):
        def gather_one(i, _):
            s = g_ref[0, 0, i]
            upd_ref[pl.ds(i, 1), :] = cfg_ref[pl.ds(s, 1), :]
            return _

        lax.fori_loop(0, blk, gather_one, None, unroll=8)
        acc = jnp.dot(prev_ref[...], w1_ref[...], preferred_element_type=jnp.float32)
        acc += jnp.dot(upd_ref[...], w2_ref[...], preferred_element_type=jnp.float32)
        out_ref[...] = jnp.maximum(acc + b_ref[...], 0.0)

    return pl.pallas_call(
        body,
        grid=(nt // blk,),
        in_specs=[
            pl.BlockSpec((1, 1, blk), lambda i: (i, 0, 0),
                         memory_space=pltpu.SMEM),
            pl.BlockSpec((blk, d), lambda i: (i, 0)),
            pl.BlockSpec((nc, d), lambda i: (0, 0)),
            pl.BlockSpec((d, d), lambda i: (0, 0)),
            pl.BlockSpec((d, d), lambda i: (0, 0)),
            pl.BlockSpec((1, d), lambda i: (0, 0)),
        ],
        out_specs=pl.BlockSpec((blk, d), lambda i: (i, 0)),
        out_shape=jax.ShapeDtypeStruct((nt, d), jnp.float32),
        scratch_shapes=[pltpu.VMEM((blk, d), jnp.float32)],
        compiler_params=pltpu.CompilerParams(
            vmem_limit_bytes=60 * 1024 * 1024),
    )(g_row, prev2d, cfg_ext, w1, w2, b2d)


def kernel(nr_ast_nodes, prev_nodes_occurrences, new_cfg_nodes_encodings,
           mapping_value_indices, mapping_key_indices, path_node_indices, W, b):
    p, l, d = prev_nodes_occurrences.shape
    n_cfg = new_cfg_nodes_encodings.shape[0]
    nt = p * l

    key_i32 = jnp.minimum(mapping_key_indices, nr_ast_nodes - 1).astype(jnp.int32)
    val_i32 = mapping_value_indices.astype(jnp.int32)
    nt_pad = -(-nt // (_NW * 128)) * _NW * 128
    path_pad = jnp.pad(path_node_indices.reshape(nt).astype(jnp.int32),
                       (0, nt_pad - nt))
    cfg_ext = jnp.concatenate(
        [new_cfg_nodes_encodings,
         jnp.zeros((8, d), new_cfg_nodes_encodings.dtype)], axis=0)

    src_map = _build_src_map(key_i32, val_i32, _N_AST, n_cfg)
    g2d = _lookup_rows(src_map, path_pad)
    g_row = g2d.reshape(nt_pad)[:nt].reshape(nt // 2000, 1, 2000)

    prev2d = prev_nodes_occurrences.reshape(nt, d)
    out2d = _gather_cat_project(prev2d, g_row, cfg_ext, W[:d], W[d:],
                                b.reshape(1, d))
    return out2d.reshape(p, l, d)
